# baseline (speedup 1.0000x reference)
import functools

import jax
import jax.numpy as jnp
from jax.experimental import pallas as pl
from jax.experimental.pallas import tpu as pltpu

X_DIM, E_DIM, G_DIM = 128, 16, 32
NUM_EMB, NUM_LAYERS, NUM_HEADS = 64, 3, 2
N_NODES, N_EDGES, N_GRAPHS = 10000, 320000, 64


def _mlp3_block(x_ref, w1, b1, w2, b2, w3, b3, o_ref):
    h = jnp.dot(x_ref[...], w1[...], preferred_element_type=jnp.float32) + b1[...]
    h = jnp.where(h > 0, h, 0.01 * h)
    h = jnp.dot(h, w2[...], preferred_element_type=jnp.float32) + b2[...]
    h = jnp.where(h > 0, h, 0.01 * h)
    o_ref[...] = jnp.dot(h, w3[...], preferred_element_type=jnp.float32) + b3[...]


def _mlp3_pallas(x, layers, block_rows):
    (w1, b1), (w2, b2), (w3, b3) = layers
    n = x.shape[0]
    assert n % block_rows == 0
    grid = n // block_rows
    full = lambda s: pl.BlockSpec(s, lambda i: (0, 0))
    return pl.pallas_call(
        _mlp3_block,
        grid=(grid,),
        in_specs=[
            pl.BlockSpec((block_rows, x.shape[1]), lambda i: (i, 0)),
            full(w1.shape), pl.BlockSpec((1, b1.shape[0]), lambda i: (0, 0)),
            full(w2.shape), pl.BlockSpec((1, b2.shape[0]), lambda i: (0, 0)),
            full(w3.shape), pl.BlockSpec((1, b3.shape[0]), lambda i: (0, 0)),
        ],
        out_specs=pl.BlockSpec((block_rows, w3.shape[1]), lambda i: (i, 0)),
        out_shape=jax.ShapeDtypeStruct((n, w3.shape[1]), jnp.float32),
    )(x, w1, b1[None], w2, b2[None], w3, b3[None])


def _graph_layer_norm(x, batch, num_graphs, eps=1e-5):
    d = x.shape[1]
    cnt = jax.ops.segment_sum(jnp.ones((x.shape[0],), x.dtype), batch, num_segments=num_graphs)
    norm = jnp.maximum(cnt, 1.0) * d
    mean = jax.ops.segment_sum(x, batch, num_segments=num_graphs).sum(axis=-1) / norm
    x = x - mean[batch][:, None]
    var = jax.ops.segment_sum(x * x, batch, num_segments=num_graphs).sum(axis=-1) / norm
    return x / jnp.sqrt(var + eps)[batch][:, None]


def _segment_softmax(alpha, index, num_segments):
    amax = jax.ops.segment_max(alpha, index, num_segments=num_segments)
    amax = jnp.where(jnp.isneginf(amax), 0.0, amax)
    ex = jnp.exp(alpha - amax[index])
    den = jax.ops.segment_sum(ex, index, num_segments=num_segments)
    return ex / (den[index] + 1e-16)


def kernel(x, edge_attr, cond, params, edge_index, batch):
    N = x.shape[0]
    G = cond.shape[0]
    H, C = NUM_HEADS, NUM_EMB
    o = _mlp3_pallas(x, params['x2h'], 1000)
    e = _mlp3_pallas(edge_attr, params['e2h'], 2000)
    c = _mlp3_pallas(cond, params['c2h'], 64)
    u = jnp.arange(N, dtype=edge_index.dtype)
    v = batch.astype(edge_index.dtype) + N
    aug_ei = jnp.concatenate([edge_index, jnp.stack([u, v]), jnp.stack([v, u])], axis=1)
    e_p = jnp.zeros((2 * N, e.shape[1]), e.dtype).at[:, 0].set(1.0)
    aug_e = jnp.concatenate([e, e_p], axis=0)
    Naug = N + G
    deg = jax.ops.segment_sum(jnp.ones((aug_ei.shape[1],), e.dtype), aug_ei[1], num_segments=Naug)
    loop_attr = jax.ops.segment_sum(aug_e, aug_ei[1], num_segments=Naug) / jnp.maximum(deg, 1.0)[:, None]
    loops = jnp.arange(Naug, dtype=edge_index.dtype)
    aug_ei = jnp.concatenate([aug_ei, jnp.stack([loops, loops])], axis=1)
    aug_e = jnp.concatenate([aug_e, loop_attr], axis=0)
    aug_batch = jnp.concatenate([batch, jnp.arange(G, dtype=batch.dtype)])
    o = jnp.concatenate([o, c], axis=0)
    src, dst = aug_ei[0], aug_ei[1]
    for lp in params['layers']:
        msg = jax.nn.relu(o[src] + aug_e) + 1e-7
        agg = jax.ops.segment_sum(msg, dst, num_segments=Naug)
        agg = (agg + o) @ lp['gen_W'] + lp['gen_b']
        x_in = jnp.concatenate([o, agg], axis=1)
        q = (x_in @ lp['Wq'] + lp['bq']).reshape(Naug, H, C)
        k = (x_in @ lp['Wk'] + lp['bk']).reshape(Naug, H, C)
        val = (x_in @ lp['Wv'] + lp['bv']).reshape(Naug, H, C)
        eh = (aug_e @ lp['We'] + lp['be']).reshape(-1, H, C)
        k_j = k[src] + eh
        alpha = (q[dst] * k_j).sum(-1) / (C ** 0.5)
        alpha = _segment_softmax(alpha, dst, Naug)
        m = (val[src] + eh) * alpha[:, :, None]
        t_out = jax.ops.segment_sum(m, dst, num_segments=Naug).reshape(Naug, H * C)
        t_out = t_out + x_in @ lp['Ws'] + lp['bs']
        h = t_out @ lp['Wl'] + lp['bl']
        o = _graph_layer_norm(o + h, aug_batch, G)
        ff = jax.nn.leaky_relu(o @ lp['Wf1'] + lp['bf1'], 0.01) @ lp['Wf2'] + lp['bf2']
        o = _graph_layer_norm(o + ff, aug_batch, G)
    cnt = jnp.maximum(jax.ops.segment_sum(jnp.ones((N,), o.dtype), batch, num_segments=G), 1.0)
    pooled = jax.ops.segment_sum(o[:N], batch, num_segments=G) / cnt[:, None]
    glob = jnp.concatenate([pooled, o[N:], c], axis=1)
    o_final = jnp.concatenate([o[:N], c[batch]], axis=1)
    return o_final, glob
